# baseline (speedup 1.0000x reference)
import functools

import jax
import jax.numpy as jnp
from jax import lax
from jax.experimental import pallas as pl
from jax.experimental.pallas import tpu as pltpu
from jax.experimental.pallas import tpu_sc as plsc

N = 10000
NA = 10240
D = 128
DH = 64
E = 320000
CHUNK = 128
EROWS = 2560
EPAD = EROWS * CHUNK
NC, NS = 2, 16
CPT = EROWS // NS
IDXB = 40
NBLK = CPT // IDXB
STR = NA // NS
QCH = STR // CHUNK

_mesh = plsc.VectorSubcoreMesh(core_axis_name="c", subcore_axis_name="s")


def _tc_xw_body(x_ref, w_ref, y_ref):
    xw = jnp.dot(x_ref[...], w_ref[...], preferred_element_type=jnp.float32)
    y_ref[0] = xw[:, :DH]
    y_ref[1] = xw[:, DH:]


@functools.partial(
    pl.kernel,
    out_type=jax.ShapeDtypeStruct((NC * NA, DH), jnp.float32),
    mesh=_mesh,
    scratch_types=[
        pltpu.VMEM((IDXB, CHUNK), jnp.int32),
        pltpu.VMEM((IDXB, CHUNK), jnp.int32),
        pltpu.VMEM((CHUNK, DH), jnp.float32),
        pltpu.VMEM((CHUNK, DH), jnp.float32),
        pltpu.VMEM((CHUNK, DH), jnp.float32),
        pltpu.VMEM((CHUNK, DH), jnp.float32),
        pltpu.VMEM((CHUNK,), jnp.float32),
        pltpu.VMEM((STR,), jnp.float32),
        pltpu.VMEM((DH,), jnp.float32),
        pltpu.VMEM((16,), jnp.float32),
        pltpu.VMEM_SHARED((NA, DH), jnp.float32),
        pltpu.VMEM_SHARED((NA, DH), jnp.float32),
        pltpu.VMEM_SHARED((NA,), jnp.float32),
        pltpu.SemaphoreType.DMA,
        pltpu.SemaphoreType.DMA,
        pltpu.SemaphoreType.DMA,
        pltpu.SemaphoreType.DMA,
        pltpu.SemaphoreType.DMA,
        pltpu.SemaphoreType.DMA,
        pltpu.SemaphoreType.DMA,
        pltpu.SemaphoreType.DMA,
        pltpu.SemaphoreType.DMA,
    ],
    compiler_params=pltpu.CompilerParams(use_tc_tiling_on_sc=False,
                                         needs_layout_passes=False),
)
def _sc_gcn(src2d, dst2d, xw_hbm, zdeg, zacc, b_hbm, a_hbm, oh,
            sidx_v, didx_v, r0, r1, r2, r3, ones_v, dv, bv, av,
            y_sp, acc, degsp,
            dsem, g0, g1, g2, g3, s0, s1, s2, s3):
    c = lax.axis_index("c")
    s = lax.axis_index("s")
    rows = (r0, r1, r2, r3)
    gs = (g0, g1, g2, g3)
    ss = (s0, s1, s2, s3)

    @pl.when(s == 0)
    def _():
        pltpu.sync_copy(zdeg, degsp)

    pltpu.sync_copy(zacc, acc.at[pl.ds(s * STR, STR)])
    pltpu.sync_copy(b_hbm.at[pl.ds(c * DH, DH)], bv)
    pltpu.sync_copy(a_hbm, av)
    for k in range(CHUNK // 16):
        ones_v[pl.ds(k * 16, 16)] = jnp.ones((16,), jnp.float32)
    plsc.subcore_barrier()

    for bq in range(NBLK):
        pltpu.sync_copy(dst2d.at[pl.ds(s * CPT + bq * IDXB, IDXB)], didx_v)

        def dfire(j, carry):
            pltpu.async_copy(ones_v, degsp.at[didx_v.at[j]], dsem, add=True)
            return carry

        lax.fori_loop(0, IDXB, dfire, 0)

        def ddrain(j, carry):
            pltpu.make_async_copy(ones_v, degsp.at[didx_v.at[0]], dsem).wait()
            return carry

        lax.fori_loop(0, IDXB, ddrain, 0)
    plsc.subcore_barrier()

    pltpu.sync_copy(degsp.at[pl.ds(s * STR, STR)], dv)

    def newt(g, carry):
        xx = dv[pl.ds(g * 16, 16)] + 1.0
        ii = plsc.bitcast(xx, jnp.int32)
        ii = 0x5F3759DF - lax.shift_right_logical(ii, 1)
        yy = plsc.bitcast(ii, jnp.float32)
        for _ in range(3):
            yy = yy * (1.5 - 0.5 * xx * yy * yy)
        dv[pl.ds(g * 16, 16)] = yy
        return carry

    lax.fori_loop(0, STR // 16, newt, 0)

    for q in range(QCH):
        base = s * STR + q * CHUNK
        pltpu.sync_copy(xw_hbm.at[pl.ds(c * NA + base, CHUNK)], r0)

        def sgrp(g, carry):
            dd = dv[pl.ds(q * CHUNK + g * 16, 16)]
            for rr in range(16):
                row = g * 16 + rr
                d = dd[rr]
                for f in range(DH // 16):
                    r0[row, pl.ds(f * 16, 16)] = \
                        r0[row, pl.ds(f * 16, 16)] * d
            return carry

        lax.fori_loop(0, CHUNK // 16, sgrp, 0)
        pltpu.sync_copy(r0, y_sp.at[pl.ds(base, CHUNK)])
    plsc.subcore_barrier()

    def start_g(jj, kb):
        pltpu.async_copy(y_sp.at[sidx_v.at[jj]], rows[kb], gs[kb])

    def wait_g(kb):
        pltpu.make_async_copy(y_sp.at[sidx_v.at[0]], rows[kb], gs[kb]).wait()

    def start_s(jj, kb):
        pltpu.async_copy(rows[kb], acc.at[didx_v.at[jj]], ss[kb], add=True)

    def wait_s(kb):
        pltpu.make_async_copy(rows[kb], acc.at[didx_v.at[0]], ss[kb]).wait()

    for bq in range(NBLK):
        pltpu.sync_copy(src2d.at[pl.ds(s * CPT + bq * IDXB, IDXB)], sidx_v)
        pltpu.sync_copy(dst2d.at[pl.ds(s * CPT + bq * IDXB, IDXB)], didx_v)
        start_g(0, 0)
        start_g(1, 1)
        wait_g(0)
        start_s(0, 0)
        start_g(2, 2)
        wait_g(1)
        start_s(1, 1)
        start_g(3, 3)

        def cbody(i, carry):
            j = 2 + 4 * i
            for k in range(4):
                kb = (2 + k) % 4
                mb = (kb + 2) % 4
                wait_g(kb)
                start_s(j + k, kb)
                wait_s(mb)
                start_g(j + k + 2, mb)
            return carry

        lax.fori_loop(0, (IDXB - 4) // 4, cbody, 0)
        wait_g(2)
        start_s(IDXB - 2, 2)
        wait_g(3)
        start_s(IDXB - 1, 3)
        wait_s(0)
        wait_s(1)
        wait_s(2)
        wait_s(3)
    plsc.subcore_barrier()

    a_vec = av[pl.ds(0, 16)]
    for q in range(QCH):
        base = s * STR + q * CHUNK
        pltpu.sync_copy(acc.at[pl.ds(base, CHUNK)], r0)
        pltpu.sync_copy(y_sp.at[pl.ds(base, CHUNK)], r1)

        def dgrp(g, carry):
            dd = dv[pl.ds(q * CHUNK + g * 16, 16)]
            for rr in range(16):
                row = g * 16 + rr
                d = dd[rr]
                for f in range(DH // 16):
                    t = (r0[row, pl.ds(f * 16, 16)]
                         + r1[row, pl.ds(f * 16, 16)]) * d \
                        + bv[pl.ds(f * 16, 16)]
                    r2[row, pl.ds(f * 16, 16)] = \
                        jnp.where(t >= 0.0, t, a_vec * t)
            return carry

        lax.fori_loop(0, CHUNK // 16, dgrp, 0)
        pltpu.sync_copy(r2, oh.at[pl.ds(c * NA + base, CHUNK)])


def kernel(x, edge_index, W, b, prelu_a):
    src = edge_index[0]
    dst = edge_index[1]
    npad = EPAD - E
    fill = jnp.arange(npad, dtype=jnp.int32)
    src2d = jnp.concatenate([src, N + (fill % (NA - N))]).reshape(EROWS, CHUNK)
    dst2d = jnp.concatenate(
        [dst, N + ((fill * 7 + 3) % (NA - N))]).reshape(EROWS, CHUNK)

    zdeg = jnp.zeros((NA,), jnp.float32)
    zacc = jnp.zeros((STR, DH), jnp.float32)

    RB = 1000
    xw3 = pl.pallas_call(
        _tc_xw_body,
        grid=(N // RB,),
        in_specs=[
            pl.BlockSpec((RB, D), lambda i: (i, 0)),
            pl.BlockSpec((D, D), lambda i: (0, 0)),
        ],
        out_specs=pl.BlockSpec((NC, RB, DH), lambda i: (0, i, 0)),
        out_shape=jax.ShapeDtypeStruct((NC, NA, DH), jnp.float32),
    )(x, W)

    oh = _sc_gcn(src2d, dst2d, xw3.reshape(NC * NA, DH), zdeg, zacc,
                 b, jnp.broadcast_to(prelu_a, (16,)))
    out = oh.reshape(NC, NA, DH)[:, :N].transpose(1, 0, 2).reshape(N, D)
    return out


# speedup vs baseline: 1.2611x; 1.2611x over previous
import functools

import jax
import jax.numpy as jnp
from jax import lax
from jax.experimental import pallas as pl
from jax.experimental.pallas import tpu as pltpu
from jax.experimental.pallas import tpu_sc as plsc

N = 10000
NPAD = 10240
D = 128
E = 320000
CHUNK = 128
EROWS = 2560
EPAD = EROWS * CHUNK
ROWS_PER_TILE = EROWS // 32
NC, NS = 2, 16
STRIPE = NPAD // NS

_mesh = plsc.VectorSubcoreMesh(core_axis_name="c", subcore_axis_name="s")


@functools.partial(
    pl.kernel,
    out_type=jax.ShapeDtypeStruct((NC * NPAD,), jnp.float32),
    mesh=_mesh,
    scratch_types=[
        pltpu.VMEM((ROWS_PER_TILE, CHUNK), jnp.int32),
        pltpu.VMEM((CHUNK,), jnp.float32),
        pltpu.VMEM_SHARED((NPAD,), jnp.float32),
        pltpu.SemaphoreType.DMA,
    ],
)
def _sc_degree(dst2d, zdeg, degp, idx_v, ones_v, acc, dsem):
    c = lax.axis_index("c")
    s = lax.axis_index("s")
    wid = s * NC + c

    @pl.when(s == 0)
    def _():
        pltpu.sync_copy(zdeg, acc)

    for k in range(CHUNK // 16):
        ones_v[pl.ds(k * 16, 16)] = jnp.ones((16,), jnp.float32)
    pltpu.sync_copy(dst2d.at[pl.ds(wid * ROWS_PER_TILE, ROWS_PER_TILE)], idx_v)
    plsc.subcore_barrier()

    def fire(j, carry):
        pltpu.async_copy(ones_v, acc.at[idx_v.at[j]], dsem, add=True)
        return carry

    lax.fori_loop(0, ROWS_PER_TILE, fire, 0)

    def drain(j, carry):
        pltpu.make_async_copy(ones_v, acc.at[idx_v.at[0]], dsem).wait()
        return carry

    lax.fori_loop(0, ROWS_PER_TILE, drain, 0)
    plsc.subcore_barrier()
    pltpu.sync_copy(acc.at[pl.ds(s * STRIPE, STRIPE)],
                    degp.at[pl.ds(c * NPAD + s * STRIPE, STRIPE)])


def _tc_xw_body(x_ref, w_ref, d0_ref, d1_ref, y_ref, dinv_ref):
    deg = d0_ref[...] + d1_ref[...] + 1.0
    dinv = lax.rsqrt(deg)
    xw = jnp.dot(x_ref[...], w_ref[...], preferred_element_type=jnp.float32)
    y_ref[...] = xw * dinv
    dinv_ref[...] = dinv


IDXB = 16
NBLK = ROWS_PER_TILE // IDXB


@functools.partial(
    pl.kernel,
    out_type=jax.ShapeDtypeStruct((NC * NPAD, D), jnp.float32),
    mesh=_mesh,
    scratch_types=[
        pltpu.VMEM((IDXB, CHUNK), jnp.int32),
        pltpu.VMEM((IDXB, CHUNK), jnp.int32),
        pltpu.VMEM((CHUNK, D), jnp.float32),
        pltpu.VMEM((CHUNK, D), jnp.float32),
        pltpu.VMEM_SHARED((NPAD, D), jnp.float32),
        pltpu.SemaphoreType.DMA,
        pltpu.SemaphoreType.DMA,
    ],
)
def _sc_messages(y_hbm, src2d, dst2d, zbig, out_hbm,
                 sidx_v, didx_v, rows_a, rows_b, acc, gs0, gs1):
    c = lax.axis_index("c")
    s = lax.axis_index("s")
    wid = s * NC + c

    pltpu.sync_copy(zbig, acc.at[pl.ds(s * STRIPE, STRIPE)])
    plsc.subcore_barrier()

    for blk in range(NBLK):
        base = wid * ROWS_PER_TILE + blk * IDXB
        pltpu.sync_copy(src2d.at[pl.ds(base, IDXB)], sidx_v)
        pltpu.sync_copy(dst2d.at[pl.ds(base, IDXB)], didx_v)
        pltpu.async_copy(y_hbm.at[sidx_v.at[0]], rows_a, gs0)
        pltpu.async_copy(y_hbm.at[sidx_v.at[1]], rows_b, gs1)

        def body(i, carry):
            j0 = 2 * i
            pltpu.make_async_copy(y_hbm.at[sidx_v.at[j0]], rows_a, gs0).wait()
            pltpu.sync_copy(rows_a, acc.at[didx_v.at[j0]], add=True)
            pltpu.async_copy(y_hbm.at[sidx_v.at[j0 + 2]], rows_a, gs0)
            pltpu.make_async_copy(y_hbm.at[sidx_v.at[j0 + 1]], rows_b,
                                  gs1).wait()
            pltpu.sync_copy(rows_b, acc.at[didx_v.at[j0 + 1]], add=True)
            pltpu.async_copy(y_hbm.at[sidx_v.at[j0 + 3]], rows_b, gs1)
            return carry

        lax.fori_loop(0, IDXB // 2 - 1, body, 0)
        j0 = IDXB - 2
        pltpu.make_async_copy(y_hbm.at[sidx_v.at[j0]], rows_a, gs0).wait()
        pltpu.sync_copy(rows_a, acc.at[didx_v.at[j0]], add=True)
        pltpu.make_async_copy(y_hbm.at[sidx_v.at[j0 + 1]], rows_b, gs1).wait()
        pltpu.sync_copy(rows_b, acc.at[didx_v.at[j0 + 1]], add=True)

    plsc.subcore_barrier()
    pltpu.sync_copy(acc.at[pl.ds(s * STRIPE, STRIPE)],
                    out_hbm.at[pl.ds(c * NPAD + s * STRIPE, STRIPE)])


def _tc_out_body(p_ref, y_ref, dinv_ref, b_ref, a_ref, o_ref):
    pp = p_ref[...]
    t = (pp[0] + pp[1] + y_ref[...]) * dinv_ref[...] + b_ref[...]
    a = a_ref[0, 0]
    o_ref[...] = jnp.where(t >= 0, t, a * t)


def kernel(x, edge_index, W, b, prelu_a):
    src = edge_index[0]
    dst = edge_index[1]
    npad = EPAD - E
    fill = jnp.arange(npad, dtype=jnp.int32)
    src_p = jnp.concatenate([src, fill % N]).reshape(EROWS, CHUNK)
    dst_p = jnp.concatenate([dst, N + (fill % (NPAD - N))]).reshape(EROWS,
                                                                    CHUNK)

    zdeg = jnp.zeros((NPAD,), jnp.float32)
    zbig = jnp.zeros((STRIPE, D), jnp.float32)

    degp = _sc_degree(dst_p, zdeg)
    d0 = degp[:N].reshape(N, 1)
    d1 = degp[NPAD:NPAD + N].reshape(N, 1)

    RB = 1000
    grid = N // RB
    y, dinv = pl.pallas_call(
        _tc_xw_body,
        grid=(grid,),
        in_specs=[
            pl.BlockSpec((RB, D), lambda i: (i, 0)),
            pl.BlockSpec((D, D), lambda i: (0, 0)),
            pl.BlockSpec((RB, 1), lambda i: (i, 0)),
            pl.BlockSpec((RB, 1), lambda i: (i, 0)),
        ],
        out_specs=[
            pl.BlockSpec((RB, D), lambda i: (i, 0)),
            pl.BlockSpec((RB, 1), lambda i: (i, 0)),
        ],
        out_shape=[
            jax.ShapeDtypeStruct((N, D), jnp.float32),
            jax.ShapeDtypeStruct((N, 1), jnp.float32),
        ],
    )(x, W, d0, d1)

    p = _sc_messages(y, src_p, dst_p, zbig).reshape(NC, NPAD, D)

    out = pl.pallas_call(
        _tc_out_body,
        grid=(grid,),
        in_specs=[
            pl.BlockSpec((NC, RB, D), lambda i: (0, i, 0)),
            pl.BlockSpec((RB, D), lambda i: (i, 0)),
            pl.BlockSpec((RB, 1), lambda i: (i, 0)),
            pl.BlockSpec((1, D), lambda i: (0, 0)),
            pl.BlockSpec((1, 1), lambda i: (0, 0)),
        ],
        out_specs=pl.BlockSpec((RB, D), lambda i: (i, 0)),
        out_shape=jax.ShapeDtypeStruct((N, D), jnp.float32),
    )(p, y, dinv, b.reshape(1, D), prelu_a.reshape(1, 1))
    return out


# speedup vs baseline: 1.3294x; 1.0542x over previous
import functools

import jax
import jax.numpy as jnp
from jax import lax
from jax.experimental import pallas as pl
from jax.experimental.pallas import tpu as pltpu
from jax.experimental.pallas import tpu_sc as plsc

N = 10000
NPAD = 10240
D = 128
E = 320000
CHUNK = 128
EROWS = 2560
EPAD = EROWS * CHUNK
ROWS_PER_TILE = EROWS // 32
NC, NS = 2, 16
STRIPE = NPAD // NS

_mesh = plsc.VectorSubcoreMesh(core_axis_name="c", subcore_axis_name="s")


@functools.partial(
    pl.kernel,
    out_type=jax.ShapeDtypeStruct((NC * NPAD,), jnp.float32),
    mesh=_mesh,
    scratch_types=[
        pltpu.VMEM((ROWS_PER_TILE, CHUNK), jnp.int32),
        pltpu.VMEM((CHUNK,), jnp.float32),
        pltpu.VMEM_SHARED((NPAD,), jnp.float32),
        pltpu.SemaphoreType.DMA,
    ],
)
def _sc_degree(dst2d, zdeg, degp, idx_v, ones_v, acc, dsem):
    c = lax.axis_index("c")
    s = lax.axis_index("s")
    wid = s * NC + c

    @pl.when(s == 0)
    def _():
        pltpu.sync_copy(zdeg, acc)

    for k in range(CHUNK // 16):
        ones_v[pl.ds(k * 16, 16)] = jnp.ones((16,), jnp.float32)
    pltpu.sync_copy(dst2d.at[pl.ds(wid * ROWS_PER_TILE, ROWS_PER_TILE)], idx_v)
    plsc.subcore_barrier()

    def fire(j, carry):
        pltpu.async_copy(ones_v, acc.at[idx_v.at[j]], dsem, add=True)
        return carry

    lax.fori_loop(0, ROWS_PER_TILE, fire, 0)

    def drain(j, carry):
        pltpu.make_async_copy(ones_v, acc.at[idx_v.at[0]], dsem).wait()
        return carry

    lax.fori_loop(0, ROWS_PER_TILE, drain, 0)
    plsc.subcore_barrier()
    pltpu.sync_copy(acc.at[pl.ds(s * STRIPE, STRIPE)],
                    degp.at[pl.ds(c * NPAD + s * STRIPE, STRIPE)])


def _tc_xw_body(x_ref, w_ref, d0_ref, d1_ref, y_ref, dinv_ref):
    deg = d0_ref[...] + d1_ref[...] + 1.0
    dinv = lax.rsqrt(deg)
    xw = jnp.dot(x_ref[...], w_ref[...], preferred_element_type=jnp.float32)
    y_ref[...] = xw * dinv
    dinv_ref[...] = dinv


IDXB = 40
NBLK = ROWS_PER_TILE // IDXB


@functools.partial(
    pl.kernel,
    out_type=jax.ShapeDtypeStruct((NC * NPAD, D), jnp.float32),
    mesh=_mesh,
    scratch_types=[
        pltpu.VMEM((IDXB, CHUNK), jnp.int32),
        pltpu.VMEM((IDXB, CHUNK), jnp.int32),
        pltpu.VMEM((CHUNK, D), jnp.float32),
        pltpu.VMEM((CHUNK, D), jnp.float32),
        pltpu.VMEM_SHARED((NPAD, D), jnp.float32),
        pltpu.SemaphoreType.DMA,
        pltpu.SemaphoreType.DMA,
    ],
)
def _sc_messages(y_hbm, src2d, dst2d, zbig, out_hbm,
                 sidx_v, didx_v, rows_a, rows_b, acc, gs0, gs1):
    c = lax.axis_index("c")
    s = lax.axis_index("s")
    wid = s * NC + c

    pltpu.sync_copy(zbig, acc.at[pl.ds(s * STRIPE, STRIPE)])
    plsc.subcore_barrier()

    for blk in range(NBLK):
        base = wid * ROWS_PER_TILE + blk * IDXB
        pltpu.sync_copy(src2d.at[pl.ds(base, IDXB)], sidx_v)
        pltpu.sync_copy(dst2d.at[pl.ds(base, IDXB)], didx_v)
        pltpu.async_copy(y_hbm.at[sidx_v.at[0]], rows_a, gs0)
        pltpu.async_copy(y_hbm.at[sidx_v.at[1]], rows_b, gs1)

        def body(i, carry):
            j0 = 2 * i
            pltpu.make_async_copy(y_hbm.at[sidx_v.at[j0]], rows_a, gs0).wait()
            pltpu.sync_copy(rows_a, acc.at[didx_v.at[j0]], add=True)
            pltpu.async_copy(y_hbm.at[sidx_v.at[j0 + 2]], rows_a, gs0)
            pltpu.make_async_copy(y_hbm.at[sidx_v.at[j0 + 1]], rows_b,
                                  gs1).wait()
            pltpu.sync_copy(rows_b, acc.at[didx_v.at[j0 + 1]], add=True)
            pltpu.async_copy(y_hbm.at[sidx_v.at[j0 + 3]], rows_b, gs1)
            return carry

        lax.fori_loop(0, IDXB // 2 - 1, body, 0)
        j0 = IDXB - 2
        pltpu.make_async_copy(y_hbm.at[sidx_v.at[j0]], rows_a, gs0).wait()
        pltpu.sync_copy(rows_a, acc.at[didx_v.at[j0]], add=True)
        pltpu.make_async_copy(y_hbm.at[sidx_v.at[j0 + 1]], rows_b, gs1).wait()
        pltpu.sync_copy(rows_b, acc.at[didx_v.at[j0 + 1]], add=True)

    plsc.subcore_barrier()
    pltpu.sync_copy(acc.at[pl.ds(s * STRIPE, STRIPE)],
                    out_hbm.at[pl.ds(c * NPAD + s * STRIPE, STRIPE)])


def _tc_out_body(p_ref, y_ref, dinv_ref, b_ref, a_ref, o_ref):
    pp = p_ref[...]
    t = (pp[0] + pp[1] + y_ref[...]) * dinv_ref[...] + b_ref[...]
    a = a_ref[0, 0]
    o_ref[...] = jnp.where(t >= 0, t, a * t)


def kernel(x, edge_index, W, b, prelu_a):
    src = edge_index[0]
    dst = edge_index[1]
    npad = EPAD - E
    fill = jnp.arange(npad, dtype=jnp.int32)
    src_p = jnp.concatenate([src, fill % N]).reshape(EROWS, CHUNK)
    dst_p = jnp.concatenate([dst, N + (fill % (NPAD - N))]).reshape(EROWS,
                                                                    CHUNK)

    zdeg = jnp.zeros((NPAD,), jnp.float32)
    zbig = jnp.zeros((STRIPE, D), jnp.float32)

    degp = _sc_degree(dst_p, zdeg)
    d0 = degp[:N].reshape(N, 1)
    d1 = degp[NPAD:NPAD + N].reshape(N, 1)

    RB = 1000
    grid = N // RB
    y, dinv = pl.pallas_call(
        _tc_xw_body,
        grid=(grid,),
        in_specs=[
            pl.BlockSpec((RB, D), lambda i: (i, 0)),
            pl.BlockSpec((D, D), lambda i: (0, 0)),
            pl.BlockSpec((RB, 1), lambda i: (i, 0)),
            pl.BlockSpec((RB, 1), lambda i: (i, 0)),
        ],
        out_specs=[
            pl.BlockSpec((RB, D), lambda i: (i, 0)),
            pl.BlockSpec((RB, 1), lambda i: (i, 0)),
        ],
        out_shape=[
            jax.ShapeDtypeStruct((N, D), jnp.float32),
            jax.ShapeDtypeStruct((N, 1), jnp.float32),
        ],
    )(x, W, d0, d1)

    p = _sc_messages(y, src_p, dst_p, zbig).reshape(NC, NPAD, D)

    out = pl.pallas_call(
        _tc_out_body,
        grid=(grid,),
        in_specs=[
            pl.BlockSpec((NC, RB, D), lambda i: (0, i, 0)),
            pl.BlockSpec((RB, D), lambda i: (i, 0)),
            pl.BlockSpec((RB, 1), lambda i: (i, 0)),
            pl.BlockSpec((1, D), lambda i: (0, 0)),
            pl.BlockSpec((1, 1), lambda i: (0, 0)),
        ],
        out_specs=pl.BlockSpec((RB, D), lambda i: (i, 0)),
        out_shape=jax.ShapeDtypeStruct((N, D), jnp.float32),
    )(p, y, dinv, b.reshape(1, D), prelu_a.reshape(1, 1))
    return out
